# initial kernel scaffold (unmeasured)
import jax
import jax.numpy as jnp
from jax import lax
from jax.experimental import pallas as pl
from jax.experimental.pallas import tpu as pltpu

N_DEV = 4
HQ = 8
DH = 128
SQ = 256
SKV_SHARD = 4096
D_MODEL = HQ * DH
BLK = 64
SCALE = 0.08838834764831843
PAYLOAD = D_MODEL + 128


def kernel(x, Wq, K_ext, V_ext, Wo):
    x2 = x.reshape(SQ, D_MODEL)
    K2 = K_ext.reshape(SKV_SHARD, D_MODEL)
    V2 = V_ext.reshape(SKV_SHARD, D_MODEL)

    def body(x_ref, wq_ref, k_ref, v_ref, wo_ref, out_ref,
             buf, send_sems, recv_sems):
        my = lax.axis_index("i")
        left = lax.rem(my + N_DEV - 1, N_DEV)
        right = lax.rem(my + 1, N_DEV)

        barrier = pltpu.get_barrier_semaphore()
        for nbr in (left, right):
            pl.semaphore_signal(
                barrier, inc=1,
                device_id=(nbr,), device_id_type=pl.DeviceIdType.MESH,
            )
        pl.semaphore_wait(barrier, 2)

        Q = jnp.dot(x_ref[...], wq_ref[...],
                    preferred_element_type=jnp.float32)

        q_blk = lax.broadcasted_iota(jnp.int32, (SQ, SKV_SHARD), 0) // BLK
        k_blk = (lax.broadcasted_iota(jnp.int32, (SQ, SKV_SHARD), 1) // BLK
                 + my * (SKV_SHARD // BLK))
        mask = (q_blk == k_blk) | (k_blk == 0) | (((q_blk + k_blk) % 3) == 0)

        ctxs = []
        sums = []
        for h in range(HQ):
            qh = Q[:, h * DH:(h + 1) * DH]
            kh = k_ref[:, h * DH:(h + 1) * DH]
            scores = lax.dot_general(
                qh, kh, (((1,), (1,)), ((), ())),
                preferred_element_type=jnp.float32) * SCALE
            w = jnp.where(mask, jnp.exp(scores), 0.0)
            ctxs.append(jnp.dot(w, v_ref[:, h * DH:(h + 1) * DH],
                                preferred_element_type=jnp.float32))
            sums.append(jnp.sum(w, axis=1, keepdims=True))
        pad = jnp.zeros((SQ, 128 - HQ), jnp.float32)
        buf[0] = jnp.concatenate(ctxs + sums + [pad], axis=1)

        for hop in range(N_DEV - 1):
            rdma = pltpu.make_async_remote_copy(
                src_ref=buf.at[hop],
                dst_ref=buf.at[hop + 1],
                send_sem=send_sems.at[hop],
                recv_sem=recv_sems.at[hop],
                device_id=(right,),
                device_id_type=pl.DeviceIdType.MESH,
            )
            rdma.start()
            rdma.wait()

        tot = buf[0] + buf[1] + buf[2] + buf[3]
        s_tot = tot[:, D_MODEL:D_MODEL + HQ]
        attn = jnp.concatenate(
            [tot[:, h * DH:(h + 1) * DH] / s_tot[:, h:h + 1]
             for h in range(HQ)], axis=1)
        out_ref[...] = jnp.dot(attn, wo_ref[...],
                               preferred_element_type=jnp.float32)

    out = pl.pallas_call(
        body,
        out_shape=jax.ShapeDtypeStruct((SQ, D_MODEL), jnp.float32),
        in_specs=[pl.BlockSpec(memory_space=pltpu.VMEM)] * 5,
        out_specs=pl.BlockSpec(memory_space=pltpu.VMEM),
        scratch_shapes=[
            pltpu.VMEM((N_DEV, SQ, PAYLOAD), jnp.float32),
            pltpu.SemaphoreType.DMA((N_DEV - 1,)),
            pltpu.SemaphoreType.DMA((N_DEV - 1,)),
        ],
        compiler_params=pltpu.CompilerParams(collective_id=0),
    )(x2, Wq, K2, V2, Wo)
    return out.reshape(1, SQ, D_MODEL)


# baseline (device time: 106657 ns/iter reference)
import jax
import jax.numpy as jnp
from jax import lax
from jax.experimental import pallas as pl
from jax.experimental.pallas import tpu as pltpu

N_DEV = 4
HQ = 8
DH = 128
SQ = 256
SKV_SHARD = 4096
D_MODEL = HQ * DH
BLK = 64
SCALE = 0.08838834764831843
PAYLOAD = D_MODEL + 128


def kernel(x, Wq, K_ext, V_ext, Wo):
    x2 = x.reshape(SQ, D_MODEL)
    K2 = K_ext.reshape(SKV_SHARD, D_MODEL)
    V2 = V_ext.reshape(SKV_SHARD, D_MODEL)

    def body(x_ref, wq_ref, k_ref, v_ref, wo_ref, out_ref,
             buf, send_sems, recv_sems):
        my = lax.axis_index("i")
        left = lax.rem(my + N_DEV - 1, N_DEV)
        right = lax.rem(my + 1, N_DEV)

        barrier = pltpu.get_barrier_semaphore()
        for nbr in (left, right):
            pl.semaphore_signal(
                barrier, inc=1,
                device_id=(nbr,), device_id_type=pl.DeviceIdType.MESH,
            )
        pl.semaphore_wait(barrier, 2)

        Q = jnp.dot(x_ref[...], wq_ref[...],
                    preferred_element_type=jnp.float32)

        q_blk = lax.broadcasted_iota(jnp.int32, (SQ, SKV_SHARD), 0) // BLK
        k_blk = (lax.broadcasted_iota(jnp.int32, (SQ, SKV_SHARD), 1) // BLK
                 + my * (SKV_SHARD // BLK))
        mask = (q_blk == k_blk) | (k_blk == 0) | (((q_blk + k_blk) % 3) == 0)

        ctxs = []
        sums = []
        for h in range(HQ):
            qh = Q[:, h * DH:(h + 1) * DH]
            kh = k_ref[:, h * DH:(h + 1) * DH]
            scores = lax.dot_general(
                qh, kh, (((1,), (1,)), ((), ())),
                preferred_element_type=jnp.float32) * SCALE
            w = jnp.where(mask, jnp.exp(scores), 0.0)
            ctxs.append(jnp.dot(w, v_ref[:, h * DH:(h + 1) * DH],
                                preferred_element_type=jnp.float32))
            sums.append(jnp.sum(w, axis=1, keepdims=True))
        pad = jnp.zeros((SQ, 128 - HQ), jnp.float32)
        buf[0] = jnp.concatenate(ctxs + sums + [pad], axis=1)

        for hop in range(N_DEV - 1):
            rdma = pltpu.make_async_remote_copy(
                src_ref=buf.at[hop],
                dst_ref=buf.at[hop + 1],
                send_sem=send_sems.at[hop],
                recv_sem=recv_sems.at[hop],
                device_id=(right,),
                device_id_type=pl.DeviceIdType.MESH,
            )
            rdma.start()
            rdma.wait()

        tot = buf[0] + buf[1] + buf[2] + buf[3]
        s_tot = tot[:, D_MODEL:D_MODEL + HQ]
        attn = jnp.concatenate(
            [tot[:, h * DH:(h + 1) * DH] / s_tot[:, h:h + 1]
             for h in range(HQ)], axis=1)
        out_ref[...] = jnp.dot(attn, wo_ref[...],
                               preferred_element_type=jnp.float32)

    out = pl.pallas_call(
        body,
        out_shape=jax.ShapeDtypeStruct((SQ, D_MODEL), jnp.float32),
        in_specs=[pl.BlockSpec(memory_space=pltpu.VMEM)] * 5,
        out_specs=pl.BlockSpec(memory_space=pltpu.VMEM),
        scratch_shapes=[
            pltpu.VMEM((N_DEV, SQ, PAYLOAD), jnp.float32),
            pltpu.SemaphoreType.DMA((N_DEV - 1,)),
            pltpu.SemaphoreType.DMA((N_DEV - 1,)),
        ],
        compiler_params=pltpu.CompilerParams(
            collective_id=0, vmem_limit_bytes=100 * 1024 * 1024),
    )(x2, Wq, K2, V2, Wo)
    return out.reshape(1, SQ, D_MODEL)
